# baseline (speedup 1.0000x reference)
import functools

import numpy as np
import jax
import jax.numpy as jnp
from jax.experimental import pallas as pl
from jax.experimental.pallas import tpu as pltpu

_EPS = 1e-5


def _rot_lanes_bf16(x, shift_left):
    n = x.shape[-1]
    if shift_left % n == 0:
        return x
    xi = pltpu.bitcast(x, jnp.int32)
    ri = pltpu.roll(xi, shift=(-shift_left) % n, axis=1)
    return pltpu.bitcast(ri, jnp.bfloat16)


def _rot_lanes_f32(x, shift_left):
    n = x.shape[-1]
    if shift_left % n == 0:
        return x
    return pltpu.roll(x, shift=(-shift_left) % n, axis=1)


def _msff_body(x_ref, mask_ref,
               w1_ref, b1_ref,
               se1_ref, se2_ref,
               w2a_ref, b2a_ref,
               w2b_ref, b2b_ref,
               out_ref,
               p0_ref, p1_ref, p2_ref, p3_ref,
               *, H, W, bblk, C, Ch):
    HW = H * W
    mask = mask_ref[...]
    mL = mask[3:4, :].astype(jnp.bfloat16)
    mR = mask[5:6, :].astype(jnp.bfloat16)
    vtop = mask[1:2, :]
    vbot = mask[7:8, :]
    slabs = (p0_ref, p1_ref, p2_ref, p3_ref)

    def build_stack(act_bf, cin, slab):
        slab[0:cin, :] = _rot_lanes_bf16(act_bf, -1) * mL
        slab[cin:2 * cin, :] = act_bf
        slab[2 * cin:3 * cin, :] = _rot_lanes_bf16(act_bf, 1) * mR

    def finish(cin, cout, w, bias, slab):
        z = jnp.dot(w, slab[0:3 * cin, :],
                    preferred_element_type=jnp.float32)
        y = (z[cout:2 * cout, :]
             + _rot_lanes_f32(z[0:cout, :], -W) * vtop
             + _rot_lanes_f32(z[2 * cout:3 * cout, :], W) * vbot)
        return jnp.maximum(y + bias, 0.0)

    w1 = w1_ref[...]
    w2a = w2a_ref[...]
    w2b = w2b_ref[...]
    b1 = b1_ref[...]
    b2a = b2a_ref[...]
    b2b = b2b_ref[...]
    se1 = se1_ref[...]
    se2 = se2_ref[...]

    xs = [x_ref[b] for b in range(bblk)]
    for b in range(bblk):
        build_stack(xs[b], C, slabs[b])

    ms = []
    for b in range(bblk):
        x = xs[b].astype(jnp.float32)
        y1 = finish(C, C, w1, b1, slabs[b])
        pooled = jnp.mean(x, axis=1, keepdims=True)
        hid = jnp.maximum(
            jnp.sum(se1 * pooled, axis=0, keepdims=True), 0.0)
        att = jax.nn.sigmoid(
            jnp.sum(se2 * hid, axis=1, keepdims=True))
        ms.append((y1 * (x * att)).astype(jnp.bfloat16))

    for b in range(bblk):
        build_stack(ms[b], C, slabs[b])
    y2s = [finish(C, Ch, w2a, b2a, slabs[b]).astype(jnp.bfloat16)
           for b in range(bblk)]
    for b in range(bblk):
        build_stack(y2s[b], Ch, slabs[b])
    for b in range(bblk):
        out_ref[b] = finish(Ch, Ch, w2b, b2b, slabs[b]).astype(jnp.bfloat16)


def _flat_w3(w_oihw, scale):
    cout, cin = w_oihw.shape[:2]
    w = w_oihw.astype(jnp.float32) * scale[:, None, None, None]
    return jnp.transpose(w, (2, 0, 3, 1)).reshape(
        3 * cout, 3 * cin).astype(jnp.bfloat16)


def _fold_bn(conv_b, gamma, beta, mean, var):
    scale = gamma / jnp.sqrt(var + _EPS)
    bias = beta + (conv_b - mean) * scale
    return scale, bias[:, None]


@functools.lru_cache(maxsize=None)
def _np_border_mask(H, W):
    hh, ww = np.meshgrid(np.arange(H), np.arange(W), indexing="ij")
    rows = []
    for dy in range(3):
        for dx in range(3):
            v = ((hh + dy - 1 >= 0) & (hh + dy - 1 < H) &
                 (ww + dx - 1 >= 0) & (ww + dx - 1 < W))
            rows.append(v.reshape(-1))
    m = np.zeros((8, H * W), np.float32)
    for r in (1, 3, 5, 7):
        m[r] = rows[r]
    return m


def kernel(x_nchw, conv1_w, conv1_b, bn1_gamma, bn1_beta, bn1_mean, bn1_var,
           se_w1, se_w2,
           conv2a_w, conv2a_b, bn2a_gamma, bn2a_beta, bn2a_mean, bn2a_var,
           conv2b_w, conv2b_b, bn2b_gamma, bn2b_beta, bn2b_mean, bn2b_var):
    B, C, H, W = x_nchw.shape
    Ch, Cr, HW = C // 2, C // 4, H * W
    Bblk = 4 if B % 4 == 0 else 1
    nsteps = B // Bblk

    x = x_nchw.astype(jnp.bfloat16).reshape(B, C, HW)

    s1, b1 = _fold_bn(conv1_b, bn1_gamma, bn1_beta, bn1_mean, bn1_var)
    w1 = _flat_w3(conv1_w, s1)
    s2a, b2a = _fold_bn(conv2a_b, bn2a_gamma, bn2a_beta, bn2a_mean, bn2a_var)
    w2a = _flat_w3(conv2a_w, s2a)
    s2b, b2b = _fold_bn(conv2b_b, bn2b_gamma, bn2b_beta, bn2b_mean, bn2b_var)
    w2b = _flat_w3(conv2b_w, s2b)
    se1 = se_w1.astype(jnp.float32)
    se2 = jnp.transpose(se_w2).astype(jnp.float32)
    mask = jnp.asarray(_np_border_mask(H, W))

    def fixed(shape):
        return pl.BlockSpec(shape, lambda b: (0,) * len(shape))

    body = functools.partial(_msff_body, H=H, W=W, bblk=Bblk, C=C, Ch=Ch)
    out = pl.pallas_call(
        body,
        out_shape=jax.ShapeDtypeStruct((B, Ch, HW), jnp.bfloat16),
        grid=(nsteps,),
        in_specs=[
            pl.BlockSpec((Bblk, C, HW), lambda b: (b, 0, 0)),
            fixed((8, HW)),
            fixed((3 * C, 3 * C)), fixed((C, 1)),
            fixed((C, Cr)), fixed((C, Cr)),
            fixed((3 * Ch, 3 * C)), fixed((Ch, 1)),
            fixed((3 * Ch, 3 * Ch)), fixed((Ch, 1)),
        ],
        out_specs=pl.BlockSpec((Bblk, Ch, HW), lambda b: (b, 0, 0)),
        scratch_shapes=[pltpu.VMEM((3 * C, HW), jnp.bfloat16)
                        for _ in range(4)],
        compiler_params=pltpu.CompilerParams(
            dimension_semantics=("parallel",)),
    )(x, mask, w1, b1, se1, se2, w2a, b2a, w2b, b2b)

    return out.astype(jnp.float32).reshape(B, Ch, H, W)


# speedup vs baseline: 1.0714x; 1.0714x over previous
import functools

import numpy as np
import jax
import jax.numpy as jnp
from jax.experimental import pallas as pl
from jax.experimental.pallas import tpu as pltpu

_EPS = 1e-5


def _rot_lanes_bf16(x, shift_left):
    n = x.shape[-1]
    if shift_left % n == 0:
        return x
    xi = pltpu.bitcast(x, jnp.int32)
    ri = pltpu.roll(xi, shift=(-shift_left) % n, axis=1)
    return pltpu.bitcast(ri, jnp.bfloat16)


def _rot_lanes_f32(x, shift_left):
    n = x.shape[-1]
    if shift_left % n == 0:
        return x
    return pltpu.roll(x, shift=(-shift_left) % n, axis=1)


def _msff_body(x_ref, mask_ref,
               w1_ref, b1_ref,
               se1_ref, se2_ref,
               w2a_ref, b2a_ref,
               w2b_ref, b2b_ref,
               out_ref,
               p0_ref, p1_ref, p2_ref, p3_ref,
               *, H, W, bblk, C, Ch):
    HW = H * W
    mask = mask_ref[...]
    mL = mask[3:4, :].astype(jnp.bfloat16)
    mR = mask[5:6, :].astype(jnp.bfloat16)
    vtop = mask[1:2, :]
    vbot = mask[7:8, :]
    slabs = (p0_ref, p1_ref, p2_ref, p3_ref)

    def build_stack(act_bf, cin, slab):
        slab[0:cin, :] = _rot_lanes_bf16(act_bf, -1) * mL
        slab[cin:2 * cin, :] = act_bf
        slab[2 * cin:3 * cin, :] = _rot_lanes_bf16(act_bf, 1) * mR

    def finish(cin, cout, w, bias, slab):
        z = jnp.dot(w, slab[0:3 * cin, :],
                    preferred_element_type=jnp.float32)
        y = (z[cout:2 * cout, :]
             + _rot_lanes_f32(z[0:cout, :], -W) * vtop
             + _rot_lanes_f32(z[2 * cout:3 * cout, :], W) * vbot)
        return jnp.maximum(y + bias, 0.0)

    w1 = w1_ref[...]
    w2a = w2a_ref[...]
    w2b = w2b_ref[...]
    b1 = b1_ref[...]
    b2a = b2a_ref[...]
    b2b = b2b_ref[...]
    se1 = se1_ref[...]
    se2 = se2_ref[...]

    xs = [x_ref[b] for b in range(bblk)]
    for b in range(bblk):
        build_stack(xs[b].astype(jnp.bfloat16), C, slabs[b])

    ms = []
    for b in range(bblk):
        x = xs[b]
        y1 = finish(C, C, w1, b1, slabs[b])
        pooled = jnp.mean(x, axis=1, keepdims=True)
        hid = jnp.maximum(
            jnp.sum(se1 * pooled, axis=0, keepdims=True), 0.0)
        att = jax.nn.sigmoid(
            jnp.sum(se2 * hid, axis=1, keepdims=True))
        ms.append((y1 * (x * att)).astype(jnp.bfloat16))

    for b in range(bblk):
        build_stack(ms[b], C, slabs[b])
    y2s = [finish(C, Ch, w2a, b2a, slabs[b]).astype(jnp.bfloat16)
           for b in range(bblk)]
    for b in range(bblk):
        build_stack(y2s[b], Ch, slabs[b])
    for b in range(bblk):
        out_ref[b] = finish(Ch, Ch, w2b, b2b, slabs[b]).astype(jnp.bfloat16)


def _flat_w3(w_oihw, scale):
    cout, cin = w_oihw.shape[:2]
    w = w_oihw.astype(jnp.float32) * scale[:, None, None, None]
    return jnp.transpose(w, (2, 0, 3, 1)).reshape(
        3 * cout, 3 * cin).astype(jnp.bfloat16)


def _fold_bn(conv_b, gamma, beta, mean, var):
    scale = gamma / jnp.sqrt(var + _EPS)
    bias = beta + (conv_b - mean) * scale
    return scale, bias[:, None]


@functools.lru_cache(maxsize=None)
def _np_border_mask(H, W):
    hh, ww = np.meshgrid(np.arange(H), np.arange(W), indexing="ij")
    rows = []
    for dy in range(3):
        for dx in range(3):
            v = ((hh + dy - 1 >= 0) & (hh + dy - 1 < H) &
                 (ww + dx - 1 >= 0) & (ww + dx - 1 < W))
            rows.append(v.reshape(-1))
    m = np.zeros((8, H * W), np.float32)
    for r in (1, 3, 5, 7):
        m[r] = rows[r]
    return m


def kernel(x_nchw, conv1_w, conv1_b, bn1_gamma, bn1_beta, bn1_mean, bn1_var,
           se_w1, se_w2,
           conv2a_w, conv2a_b, bn2a_gamma, bn2a_beta, bn2a_mean, bn2a_var,
           conv2b_w, conv2b_b, bn2b_gamma, bn2b_beta, bn2b_mean, bn2b_var):
    B, C, H, W = x_nchw.shape
    Ch, Cr, HW = C // 2, C // 4, H * W
    Bblk = 4 if B % 4 == 0 else 1
    nsteps = B // Bblk

    x = x_nchw.astype(jnp.float32).reshape(B, C, HW)

    s1, b1 = _fold_bn(conv1_b, bn1_gamma, bn1_beta, bn1_mean, bn1_var)
    w1 = _flat_w3(conv1_w, s1)
    s2a, b2a = _fold_bn(conv2a_b, bn2a_gamma, bn2a_beta, bn2a_mean, bn2a_var)
    w2a = _flat_w3(conv2a_w, s2a)
    s2b, b2b = _fold_bn(conv2b_b, bn2b_gamma, bn2b_beta, bn2b_mean, bn2b_var)
    w2b = _flat_w3(conv2b_w, s2b)
    se1 = se_w1.astype(jnp.float32)
    se2 = jnp.transpose(se_w2).astype(jnp.float32)
    mask = jnp.asarray(_np_border_mask(H, W))

    def fixed(shape):
        return pl.BlockSpec(shape, lambda b: (0,) * len(shape))

    body = functools.partial(_msff_body, H=H, W=W, bblk=Bblk, C=C, Ch=Ch)
    out = pl.pallas_call(
        body,
        out_shape=jax.ShapeDtypeStruct((B, Ch, HW), jnp.bfloat16),
        grid=(nsteps,),
        in_specs=[
            pl.BlockSpec((Bblk, C, HW), lambda b: (b, 0, 0)),
            fixed((8, HW)),
            fixed((3 * C, 3 * C)), fixed((C, 1)),
            fixed((C, Cr)), fixed((C, Cr)),
            fixed((3 * Ch, 3 * C)), fixed((Ch, 1)),
            fixed((3 * Ch, 3 * Ch)), fixed((Ch, 1)),
        ],
        out_specs=pl.BlockSpec((Bblk, Ch, HW), lambda b: (b, 0, 0)),
        scratch_shapes=[pltpu.VMEM((3 * C, HW), jnp.bfloat16)
                        for _ in range(4)],
        compiler_params=pltpu.CompilerParams(
            dimension_semantics=("parallel",)),
    )(x, mask, w1, b1, se1, se2, w2a, b2a, w2b, b2b)

    return out.astype(jnp.float32).reshape(B, Ch, H, W)


# speedup vs baseline: 1.0753x; 1.0036x over previous
import functools

import numpy as np
import jax
import jax.numpy as jnp
from jax.experimental import pallas as pl
from jax.experimental.pallas import tpu as pltpu

_EPS = 1e-5


def _rot_lanes_bf16(x, shift_left):
    n = x.shape[-1]
    if shift_left % n == 0:
        return x
    xi = pltpu.bitcast(x, jnp.int32)
    ri = pltpu.roll(xi, shift=(-shift_left) % n, axis=1)
    return pltpu.bitcast(ri, jnp.bfloat16)


def _rot_lanes_f32(x, shift_left):
    n = x.shape[-1]
    if shift_left % n == 0:
        return x
    return pltpu.roll(x, shift=(-shift_left) % n, axis=1)


def _msff_body(x_ref, mask_ref,
               w1_ref, b1_ref,
               se1_ref, se2_ref,
               w2a_ref, b2a_ref,
               w2b_ref, b2b_ref,
               out_ref,
               p0_ref, p1_ref, p2_ref, p3_ref,
               *, H, W, bblk, C, Ch):
    HW = H * W
    mask = mask_ref[...]
    mL = mask[3:4, :].astype(jnp.bfloat16)
    mR = mask[5:6, :].astype(jnp.bfloat16)
    vtop_bf = mask[1:2, :].astype(jnp.bfloat16)
    vbot_bf = mask[7:8, :].astype(jnp.bfloat16)
    slabs = (p0_ref, p1_ref, p2_ref, p3_ref)

    def build_stack(act_bf, cin, slab):
        slab[0:cin, :] = _rot_lanes_bf16(act_bf, -1) * mL
        slab[cin:2 * cin, :] = act_bf
        slab[2 * cin:3 * cin, :] = _rot_lanes_bf16(act_bf, 1) * mR

    def finish(cin, cout, w, bias, slab):
        T = slab[0:3 * cin, :]
        z0 = jnp.dot(w[0:cout, :], T, preferred_element_type=jnp.float32)
        z1 = jnp.dot(w[cout:2 * cout, :], T, preferred_element_type=jnp.float32)
        z2 = jnp.dot(w[2 * cout:3 * cout, :], T, preferred_element_type=jnp.float32)
        t0 = _rot_lanes_bf16(z0.astype(jnp.bfloat16), -W) * vtop_bf
        t2 = _rot_lanes_bf16(z2.astype(jnp.bfloat16), W) * vbot_bf
        y = z1 + t0.astype(jnp.float32) + t2.astype(jnp.float32)
        return jnp.maximum(y + bias, 0.0)

    w1 = w1_ref[...]
    w2a = w2a_ref[...]
    w2b = w2b_ref[...]
    b1 = b1_ref[...]
    b2a = b2a_ref[...]
    b2b = b2b_ref[...]
    se1 = se1_ref[...]
    se2 = se2_ref[...]

    xs = [x_ref[b] for b in range(bblk)]
    for b in range(bblk):
        build_stack(xs[b].astype(jnp.bfloat16), C, slabs[b])

    ms = []
    for b in range(bblk):
        x = xs[b]
        y1 = finish(C, C, w1, b1, slabs[b])
        pooled = jnp.mean(x, axis=1, keepdims=True)
        hid = jnp.maximum(
            jnp.sum(se1 * pooled, axis=0, keepdims=True), 0.0)
        att = jax.nn.sigmoid(
            jnp.sum(se2 * hid, axis=1, keepdims=True))
        ms.append((y1 * (x * att)).astype(jnp.bfloat16))

    for b in range(bblk):
        build_stack(ms[b], C, slabs[b])
    y2s = [finish(C, Ch, w2a, b2a, slabs[b]).astype(jnp.bfloat16)
           for b in range(bblk)]
    for b in range(bblk):
        build_stack(y2s[b], Ch, slabs[b])
    for b in range(bblk):
        out_ref[b] = finish(Ch, Ch, w2b, b2b, slabs[b]).astype(jnp.bfloat16)


def _flat_w3(w_oihw, scale):
    cout, cin = w_oihw.shape[:2]
    w = w_oihw.astype(jnp.float32) * scale[:, None, None, None]
    return jnp.transpose(w, (2, 0, 3, 1)).reshape(
        3 * cout, 3 * cin).astype(jnp.bfloat16)


def _fold_bn(conv_b, gamma, beta, mean, var):
    scale = gamma / jnp.sqrt(var + _EPS)
    bias = beta + (conv_b - mean) * scale
    return scale, bias[:, None]


@functools.lru_cache(maxsize=None)
def _np_border_mask(H, W):
    hh, ww = np.meshgrid(np.arange(H), np.arange(W), indexing="ij")
    rows = []
    for dy in range(3):
        for dx in range(3):
            v = ((hh + dy - 1 >= 0) & (hh + dy - 1 < H) &
                 (ww + dx - 1 >= 0) & (ww + dx - 1 < W))
            rows.append(v.reshape(-1))
    m = np.zeros((8, H * W), np.float32)
    for r in (1, 3, 5, 7):
        m[r] = rows[r]
    return m


def kernel(x_nchw, conv1_w, conv1_b, bn1_gamma, bn1_beta, bn1_mean, bn1_var,
           se_w1, se_w2,
           conv2a_w, conv2a_b, bn2a_gamma, bn2a_beta, bn2a_mean, bn2a_var,
           conv2b_w, conv2b_b, bn2b_gamma, bn2b_beta, bn2b_mean, bn2b_var):
    B, C, H, W = x_nchw.shape
    Ch, Cr, HW = C // 2, C // 4, H * W
    Bblk = 4 if B % 4 == 0 else 1
    nsteps = B // Bblk

    x = x_nchw.astype(jnp.float32).reshape(B, C, HW)

    s1, b1 = _fold_bn(conv1_b, bn1_gamma, bn1_beta, bn1_mean, bn1_var)
    w1 = _flat_w3(conv1_w, s1)
    s2a, b2a = _fold_bn(conv2a_b, bn2a_gamma, bn2a_beta, bn2a_mean, bn2a_var)
    w2a = _flat_w3(conv2a_w, s2a)
    s2b, b2b = _fold_bn(conv2b_b, bn2b_gamma, bn2b_beta, bn2b_mean, bn2b_var)
    w2b = _flat_w3(conv2b_w, s2b)
    se1 = se_w1.astype(jnp.float32)
    se2 = jnp.transpose(se_w2).astype(jnp.float32)
    mask = jnp.asarray(_np_border_mask(H, W))

    def fixed(shape):
        return pl.BlockSpec(shape, lambda b: (0,) * len(shape))

    body = functools.partial(_msff_body, H=H, W=W, bblk=Bblk, C=C, Ch=Ch)
    out = pl.pallas_call(
        body,
        out_shape=jax.ShapeDtypeStruct((B, Ch, HW), jnp.bfloat16),
        grid=(nsteps,),
        in_specs=[
            pl.BlockSpec((Bblk, C, HW), lambda b: (b, 0, 0)),
            fixed((8, HW)),
            fixed((3 * C, 3 * C)), fixed((C, 1)),
            fixed((C, Cr)), fixed((C, Cr)),
            fixed((3 * Ch, 3 * C)), fixed((Ch, 1)),
            fixed((3 * Ch, 3 * Ch)), fixed((Ch, 1)),
        ],
        out_specs=pl.BlockSpec((Bblk, Ch, HW), lambda b: (b, 0, 0)),
        scratch_shapes=[pltpu.VMEM((3 * C, HW), jnp.bfloat16)
                        for _ in range(4)],
        compiler_params=pltpu.CompilerParams(
            dimension_semantics=("parallel",)),
    )(x, mask, w1, b1, se1, se2, w2a, b2a, w2b, b2b)

    return out.astype(jnp.float32).reshape(B, Ch, H, W)


# speedup vs baseline: 1.1015x; 1.0244x over previous
import functools

import numpy as np
import jax
import jax.numpy as jnp
from jax.experimental import pallas as pl
from jax.experimental.pallas import tpu as pltpu

_EPS = 1e-5


def _rot_lanes_bf16(x, shift_left):
    n = x.shape[-1]
    if shift_left % n == 0:
        return x
    xi = pltpu.bitcast(x, jnp.int32)
    ri = pltpu.roll(xi, shift=(-shift_left) % n, axis=1)
    return pltpu.bitcast(ri, jnp.bfloat16)


def _rot_lanes_f32(x, shift_left):
    n = x.shape[-1]
    if shift_left % n == 0:
        return x
    return pltpu.roll(x, shift=(-shift_left) % n, axis=1)


def _msff_body(x_ref, mask_ref,
               w1_ref, b1_ref,
               se1_ref, se2_ref,
               w2a_ref, b2a_ref,
               w2b_ref, b2b_ref,
               out_ref,
               p0_ref, p1_ref, p2_ref, p3_ref,
               *, H, W, bblk, C, Ch):
    HW = H * W
    mask = mask_ref[...]
    mL = mask[3:4, :].astype(jnp.bfloat16)
    mR = mask[5:6, :].astype(jnp.bfloat16)
    vtop_bf = mask[1:2, :].astype(jnp.bfloat16)
    vbot_bf = mask[7:8, :].astype(jnp.bfloat16)
    slabs = (p0_ref, p1_ref, p2_ref, p3_ref)

    def build_stack(act_bf, cin, slab):
        slab[0:cin, :] = _rot_lanes_bf16(act_bf, -1) * mL
        slab[cin:2 * cin, :] = act_bf
        slab[2 * cin:3 * cin, :] = _rot_lanes_bf16(act_bf, 1) * mR

    def finish(cin, cout, w, bias, slab):
        z = jnp.dot(w, slab[0:3 * cin, :],
                    preferred_element_type=jnp.float32)
        t0 = _rot_lanes_bf16(z[0:cout, :].astype(jnp.bfloat16), -W) * vtop_bf
        t2 = _rot_lanes_bf16(z[2 * cout:3 * cout, :].astype(jnp.bfloat16),
                             W) * vbot_bf
        y = z[cout:2 * cout, :] + t0.astype(jnp.float32) + t2.astype(jnp.float32)
        return jnp.maximum(y + bias, 0.0)

    w1 = w1_ref[...]
    w2a = w2a_ref[...]
    w2b = w2b_ref[...]
    b1 = b1_ref[...]
    b2a = b2a_ref[...]
    b2b = b2b_ref[...]
    se1 = se1_ref[...]
    se2 = se2_ref[...]

    xs = [x_ref[b] for b in range(bblk)]
    for b in range(bblk):
        build_stack(xs[b].astype(jnp.bfloat16), C, slabs[b])

    ms = []
    for b in range(bblk):
        x = xs[b]
        y1 = finish(C, C, w1, b1, slabs[b])
        pooled = jnp.mean(x, axis=1, keepdims=True)
        hid = jnp.maximum(
            jnp.sum(se1 * pooled, axis=0, keepdims=True), 0.0)
        att = jax.nn.sigmoid(
            jnp.sum(se2 * hid, axis=1, keepdims=True))
        ms.append((y1 * (x * att)).astype(jnp.bfloat16))

    for b in range(bblk):
        build_stack(ms[b], C, slabs[b])
    y2s = [finish(C, Ch, w2a, b2a, slabs[b]).astype(jnp.bfloat16)
           for b in range(bblk)]
    for b in range(bblk):
        build_stack(y2s[b], Ch, slabs[b])
    for b in range(bblk):
        out_ref[b] = finish(Ch, Ch, w2b, b2b, slabs[b]).astype(jnp.bfloat16)


def _flat_w3(w_oihw, scale):
    cout, cin = w_oihw.shape[:2]
    w = w_oihw.astype(jnp.float32) * scale[:, None, None, None]
    return jnp.transpose(w, (2, 0, 3, 1)).reshape(
        3 * cout, 3 * cin).astype(jnp.bfloat16)


def _fold_bn(conv_b, gamma, beta, mean, var):
    scale = gamma / jnp.sqrt(var + _EPS)
    bias = beta + (conv_b - mean) * scale
    return scale, bias[:, None]


@functools.lru_cache(maxsize=None)
def _np_border_mask(H, W):
    hh, ww = np.meshgrid(np.arange(H), np.arange(W), indexing="ij")
    rows = []
    for dy in range(3):
        for dx in range(3):
            v = ((hh + dy - 1 >= 0) & (hh + dy - 1 < H) &
                 (ww + dx - 1 >= 0) & (ww + dx - 1 < W))
            rows.append(v.reshape(-1))
    m = np.zeros((8, H * W), np.float32)
    for r in (1, 3, 5, 7):
        m[r] = rows[r]
    return m


def kernel(x_nchw, conv1_w, conv1_b, bn1_gamma, bn1_beta, bn1_mean, bn1_var,
           se_w1, se_w2,
           conv2a_w, conv2a_b, bn2a_gamma, bn2a_beta, bn2a_mean, bn2a_var,
           conv2b_w, conv2b_b, bn2b_gamma, bn2b_beta, bn2b_mean, bn2b_var):
    B, C, H, W = x_nchw.shape
    Ch, Cr, HW = C // 2, C // 4, H * W
    Bblk = 4 if B % 4 == 0 else 1
    nsteps = B // Bblk

    x = x_nchw.astype(jnp.float32).reshape(B, C, HW)

    s1, b1 = _fold_bn(conv1_b, bn1_gamma, bn1_beta, bn1_mean, bn1_var)
    w1 = _flat_w3(conv1_w, s1)
    s2a, b2a = _fold_bn(conv2a_b, bn2a_gamma, bn2a_beta, bn2a_mean, bn2a_var)
    w2a = _flat_w3(conv2a_w, s2a)
    s2b, b2b = _fold_bn(conv2b_b, bn2b_gamma, bn2b_beta, bn2b_mean, bn2b_var)
    w2b = _flat_w3(conv2b_w, s2b)
    se1 = se_w1.astype(jnp.float32)
    se2 = jnp.transpose(se_w2).astype(jnp.float32)
    mask = jnp.asarray(_np_border_mask(H, W))

    def fixed(shape):
        return pl.BlockSpec(shape, lambda b: (0,) * len(shape))

    body = functools.partial(_msff_body, H=H, W=W, bblk=Bblk, C=C, Ch=Ch)
    out = pl.pallas_call(
        body,
        out_shape=jax.ShapeDtypeStruct((B, Ch, HW), jnp.bfloat16),
        grid=(nsteps,),
        in_specs=[
            pl.BlockSpec((Bblk, C, HW), lambda b: (b, 0, 0)),
            fixed((8, HW)),
            fixed((3 * C, 3 * C)), fixed((C, 1)),
            fixed((C, Cr)), fixed((C, Cr)),
            fixed((3 * Ch, 3 * C)), fixed((Ch, 1)),
            fixed((3 * Ch, 3 * Ch)), fixed((Ch, 1)),
        ],
        out_specs=pl.BlockSpec((Bblk, Ch, HW), lambda b: (b, 0, 0)),
        scratch_shapes=[pltpu.VMEM((3 * C, HW), jnp.bfloat16)
                        for _ in range(4)],
        compiler_params=pltpu.CompilerParams(
            dimension_semantics=("parallel",)),
    )(x, mask, w1, b1, se1, se2, w2a, b2a, w2b, b2b)

    return out.astype(jnp.float32).reshape(B, Ch, H, W)


# speedup vs baseline: 1.1514x; 1.0453x over previous
import functools

import numpy as np
import jax
import jax.numpy as jnp
from jax.experimental import pallas as pl
from jax.experimental.pallas import tpu as pltpu

_EPS = 1e-5


def _rot_lanes_bf16(x, shift_left):
    n = x.shape[-1]
    if shift_left % n == 0:
        return x
    xi = pltpu.bitcast(x, jnp.int32)
    ri = pltpu.roll(xi, shift=(-shift_left) % n, axis=1)
    return pltpu.bitcast(ri, jnp.bfloat16)


def _rot_lanes_f32(x, shift_left):
    n = x.shape[-1]
    if shift_left % n == 0:
        return x
    return pltpu.roll(x, shift=(-shift_left) % n, axis=1)


def _msff_body(x_ref, mask_ref,
               w1_ref, b1_ref,
               se1_ref, se2_ref,
               w2a_ref, b2a_ref,
               w2b_ref, b2b_ref,
               out_ref,
               p0_ref, p1_ref, p2_ref, p3_ref, p4_ref, p5_ref, p6_ref, p7_ref,
               *, H, W, bblk, C, Ch):
    HW = H * W
    mask = mask_ref[...]
    mL = mask[3:4, :].astype(jnp.bfloat16)
    mR = mask[5:6, :].astype(jnp.bfloat16)
    vtop_bf = mask[1:2, :].astype(jnp.bfloat16)
    vbot_bf = mask[7:8, :].astype(jnp.bfloat16)
    slabs = (p0_ref, p1_ref, p2_ref, p3_ref, p4_ref, p5_ref, p6_ref, p7_ref)

    def build_stack(act_bf, cin, slab):
        slab[0:cin, :] = _rot_lanes_bf16(act_bf, -1) * mL
        slab[cin:2 * cin, :] = act_bf
        slab[2 * cin:3 * cin, :] = _rot_lanes_bf16(act_bf, 1) * mR

    def finish(cin, cout, w, bias, slab):
        z = jnp.dot(w, slab[0:3 * cin, :],
                    preferred_element_type=jnp.float32)
        t0 = _rot_lanes_bf16(z[0:cout, :].astype(jnp.bfloat16), -W) * vtop_bf
        t2 = _rot_lanes_bf16(z[2 * cout:3 * cout, :].astype(jnp.bfloat16),
                             W) * vbot_bf
        y = z[cout:2 * cout, :] + t0.astype(jnp.float32) + t2.astype(jnp.float32)
        return jnp.maximum(y + bias, 0.0)

    w1 = w1_ref[...]
    w2a = w2a_ref[...]
    w2b = w2b_ref[...]
    b1 = b1_ref[...]
    b2a = b2a_ref[...]
    b2b = b2b_ref[...]
    se1 = se1_ref[...]
    se2 = se2_ref[...]

    xs = [x_ref[b] for b in range(bblk)]
    for b in range(bblk):
        build_stack(xs[b].astype(jnp.bfloat16), C, slabs[b])

    ms = []
    for b in range(bblk):
        x = xs[b]
        y1 = finish(C, C, w1, b1, slabs[b])
        pooled = jnp.mean(x, axis=1, keepdims=True)
        hid = jnp.maximum(
            jnp.sum(se1 * pooled, axis=0, keepdims=True), 0.0)
        att = jax.nn.sigmoid(
            jnp.sum(se2 * hid, axis=1, keepdims=True))
        ms.append((y1 * (x * att)).astype(jnp.bfloat16))

    for b in range(bblk):
        build_stack(ms[b], C, slabs[b])
    y2s = [finish(C, Ch, w2a, b2a, slabs[b]).astype(jnp.bfloat16)
           for b in range(bblk)]
    for b in range(bblk):
        build_stack(y2s[b], Ch, slabs[b])
    for b in range(bblk):
        out_ref[b] = finish(Ch, Ch, w2b, b2b, slabs[b]).astype(jnp.bfloat16)


def _flat_w3(w_oihw, scale):
    cout, cin = w_oihw.shape[:2]
    w = w_oihw.astype(jnp.float32) * scale[:, None, None, None]
    return jnp.transpose(w, (2, 0, 3, 1)).reshape(
        3 * cout, 3 * cin).astype(jnp.bfloat16)


def _fold_bn(conv_b, gamma, beta, mean, var):
    scale = gamma / jnp.sqrt(var + _EPS)
    bias = beta + (conv_b - mean) * scale
    return scale, bias[:, None]


@functools.lru_cache(maxsize=None)
def _np_border_mask(H, W):
    hh, ww = np.meshgrid(np.arange(H), np.arange(W), indexing="ij")
    rows = []
    for dy in range(3):
        for dx in range(3):
            v = ((hh + dy - 1 >= 0) & (hh + dy - 1 < H) &
                 (ww + dx - 1 >= 0) & (ww + dx - 1 < W))
            rows.append(v.reshape(-1))
    m = np.zeros((8, H * W), np.float32)
    for r in (1, 3, 5, 7):
        m[r] = rows[r]
    return m


def kernel(x_nchw, conv1_w, conv1_b, bn1_gamma, bn1_beta, bn1_mean, bn1_var,
           se_w1, se_w2,
           conv2a_w, conv2a_b, bn2a_gamma, bn2a_beta, bn2a_mean, bn2a_var,
           conv2b_w, conv2b_b, bn2b_gamma, bn2b_beta, bn2b_mean, bn2b_var):
    B, C, H, W = x_nchw.shape
    Ch, Cr, HW = C // 2, C // 4, H * W
    Bblk = 8 if B % 8 == 0 else 1
    nsteps = B // Bblk

    x = x_nchw.astype(jnp.float32).reshape(B, C, HW)

    s1, b1 = _fold_bn(conv1_b, bn1_gamma, bn1_beta, bn1_mean, bn1_var)
    w1 = _flat_w3(conv1_w, s1)
    s2a, b2a = _fold_bn(conv2a_b, bn2a_gamma, bn2a_beta, bn2a_mean, bn2a_var)
    w2a = _flat_w3(conv2a_w, s2a)
    s2b, b2b = _fold_bn(conv2b_b, bn2b_gamma, bn2b_beta, bn2b_mean, bn2b_var)
    w2b = _flat_w3(conv2b_w, s2b)
    se1 = se_w1.astype(jnp.float32)
    se2 = jnp.transpose(se_w2).astype(jnp.float32)
    mask = jnp.asarray(_np_border_mask(H, W))

    def fixed(shape):
        return pl.BlockSpec(shape, lambda b: (0,) * len(shape))

    body = functools.partial(_msff_body, H=H, W=W, bblk=Bblk, C=C, Ch=Ch)
    out = pl.pallas_call(
        body,
        out_shape=jax.ShapeDtypeStruct((B, Ch, HW), jnp.bfloat16),
        grid=(nsteps,),
        in_specs=[
            pl.BlockSpec((Bblk, C, HW), lambda b: (b, 0, 0)),
            fixed((8, HW)),
            fixed((3 * C, 3 * C)), fixed((C, 1)),
            fixed((C, Cr)), fixed((C, Cr)),
            fixed((3 * Ch, 3 * C)), fixed((Ch, 1)),
            fixed((3 * Ch, 3 * Ch)), fixed((Ch, 1)),
        ],
        out_specs=pl.BlockSpec((Bblk, Ch, HW), lambda b: (b, 0, 0)),
        scratch_shapes=[pltpu.VMEM((3 * C, HW), jnp.bfloat16)
                        for _ in range(8)],
        compiler_params=pltpu.CompilerParams(
            dimension_semantics=("parallel",)),
    )(x, mask, w1, b1, se1, se2, w2a, b2a, w2b, b2b)

    return out.astype(jnp.float32).reshape(B, Ch, H, W)
